# baseline (speedup 1.0000x reference)
import functools
import numpy as np
import jax
import jax.numpy as jnp
from jax import lax
from jax.experimental import pallas as pl
from jax.experimental.pallas import tpu as pltpu
from jax.experimental.pallas import tpu_sc as plsc

N = 10000
F = 128
NRBF = 20
CUTOFF = 5.0
NCH = 8
CW = 16
NINE = 9
CROW = 256

NC, NS, NW = 2, 16, 32
EB = 48
NPADN = 10112
ROWS_PER_TILE = NPADN // NS
NODE_PAD = 10240


def _node_mlp(s_pad, W1, b1, W2pp, b2pp):
    nb = s_pad.shape[0] // 256

    def body(s_ref, w1_ref, b1_ref, w2_ref, b2_ref, o_ref):
        h = jax.nn.silu(jnp.dot(s_ref[...], w1_ref[...],
                                preferred_element_type=jnp.float32) + b1_ref[...])
        o_ref[...] = jnp.dot(h, w2_ref[...],
                             preferred_element_type=jnp.float32) + b2_ref[...]

    return pl.pallas_call(
        body,
        grid=(nb,),
        in_specs=[
            pl.BlockSpec((256, F), lambda i: (i, 0)),
            pl.BlockSpec((F, F), lambda i: (0, 0)),
            pl.BlockSpec((1, F), lambda i: (0, 0)),
            pl.BlockSpec((F, NCH * CROW), lambda i: (0, 0)),
            pl.BlockSpec((1, NCH * CROW), lambda i: (0, 0)),
        ],
        out_specs=pl.BlockSpec((256, NCH * CROW), lambda i: (i, 0)),
        out_shape=jax.ShapeDtypeStruct((s_pad.shape[0], NCH * CROW), jnp.float32),
    )(s_pad, W1, b1.reshape(1, F), W2pp, b2pp.reshape(1, NCH * CROW))


def _edge_weights(r_pad, Wdp, bdp):
    epad = r_pad.shape[0]
    nb = epad // 512

    def body(r_ref, wd_ref, bd_ref, w_ref):
        r = r_ref[...]
        d = jnp.sqrt(jnp.sum(r * r + 1e-8, axis=1, keepdims=True))
        u3 = r / d
        u48 = jnp.concatenate(
            [jnp.broadcast_to(u3[:, kk:kk + 1], (512, CW)) for kk in range(3)],
            axis=1)
        nline = (lax.broadcasted_iota(jnp.int32, (512, NRBF), 1) + 1
                 ).astype(jnp.float32)
        rbf = jnp.sin(nline * (np.pi / CUTOFF) * d) / d
        fcut = 0.5 * (jnp.cos(d * (np.pi / CUTOFF)) + 1.0)
        fcut = jnp.where(d < CUTOFF, fcut, 0.0)
        pad = jnp.zeros((512, CROW - NINE * CW - 3 * CW), jnp.float32)
        for c in range(NCH):
            wm = (jnp.dot(rbf, wd_ref[c],
                          preferred_element_type=jnp.float32)
                  + bd_ref[c]) * fcut
            w_ref[c] = jnp.concatenate([wm, u48, pad], axis=1)

    return pl.pallas_call(
        body,
        grid=(nb,),
        in_specs=[
            pl.BlockSpec((512, 3), lambda i: (i, 0)),
            pl.BlockSpec((NCH, NRBF, NINE * CW), lambda i: (0, 0, 0)),
            pl.BlockSpec((NCH, 1, NINE * CW), lambda i: (0, 0, 0)),
        ],
        out_specs=pl.BlockSpec((NCH, 512, CROW), lambda i: (0, i, 0)),
        out_shape=jax.ShapeDtypeStruct((NCH, epad, CROW), jnp.float32),
    )(r_pad, Wdp, bdp)


def _sc_messages(inv_c, w_c, srctab, dstaux, src8, dst8, sscat, zeros):
    epad = w_c.shape[1]
    per_tile = epad // NW
    nblk = per_tile // EB
    mesh = plsc.VectorSubcoreMesh(core_axis_name="c", subcore_axis_name="s")

    @functools.partial(
        pl.kernel,
        mesh=mesh,
        out_type=jax.ShapeDtypeStruct((NC, NCH, NPADN, F), jnp.float32),
        scratch_types=[
            pltpu.VMEM((EB, CROW), jnp.float32),
            pltpu.VMEM((EB, CROW), jnp.float32),
            pltpu.VMEM((EB, F), jnp.float32),
            pltpu.VMEM((EB, F), jnp.float32),
            pltpu.VMEM((EB, F), jnp.float32),
            pltpu.VMEM((EB,), jnp.int32),
            pltpu.VMEM((EB,), jnp.int32),
            pltpu.VMEM((EB,), jnp.int32),
            pltpu.VMEM((EB,), jnp.int32),
            pltpu.VMEM((EB,), jnp.int32),
            pltpu.VMEM((EB,), jnp.int32),
            pltpu.VMEM((EB,), jnp.int32),
            pltpu.VMEM((EB,), jnp.int32),
            pltpu.VMEM((EB,), jnp.int32),
            pltpu.VMEM((EB,), jnp.int32),
            pltpu.SemaphoreType.DMA,
            pltpu.SemaphoreType.DMA,
            pltpu.VMEM_SHARED((NPADN, F), jnp.float32),
        ],
    )
    def k(inv_h, w_h, st_h, dt_h, s8_h, d8_h, ss_h, z_h,
          out_h, inv_v, w_v, sr_v, dr_v, msg_v,
          s8a, s8b, d8a, d8b, ssa, ssb, sia, sib, dia, dib,
          semg, semi, acc):
        cid = lax.axis_index("c")
        sid = lax.axis_index("s")
        wid = cid * NS + sid
        e_base = wid * per_tile
        r0 = sid * ROWS_PER_TILE
        bufs = ((s8a, d8a, ssa, sia, dia), (s8b, d8b, ssb, sib, dib))

        def load_idx(e0, bu):
            return [pltpu.async_copy(s8_h.at[pl.ds(e0, EB)], bu[0], semi),
                    pltpu.async_copy(d8_h.at[pl.ds(e0, EB)], bu[1], semi),
                    pltpu.async_copy(ss_h.at[pl.ds(e0, EB)], bu[2], semi)]

        def finish_idx(cps, bu, c):
            for cp in cps:
                cp.wait()
            for j in range(EB // 16):
                sl = pl.ds(j * 16, 16)
                bu[3][sl] = bu[0][sl] + c
                bu[4][sl] = bu[1][sl] + c

        def chunk_body(c, carry):
            pltpu.sync_copy(z_h.at[pl.ds(r0, ROWS_PER_TILE)],
                            acc.at[pl.ds(r0, ROWS_PER_TILE)])
            plsc.subcore_barrier()
            finish_idx(load_idx(e_base, bufs[0]), bufs[0], c)

            def block_body(p, bcarry):
              for half in range(2):
                b = 2 * p + half
                cur = bufs[half]
                nxt = bufs[1 - half]
                e0 = e_base + b * EB
                gps = [pltpu.async_copy(inv_h.at[cur[4]], inv_v, semg),
                       pltpu.async_copy(st_h.at[cur[3]], sr_v, semg),
                       pltpu.async_copy(dt_h.at[cur[4]], dr_v, semg),
                       pltpu.async_copy(w_h.at[c, pl.ds(e0, EB)], w_v, semg)]
                icps = load_idx(e0 + EB, nxt)
                for gp in gps:
                    gp.wait()

                def edge_body(e, ecarry):
                    s = [inv_v[e, pl.ds(j * CW, CW)] * w_v[e, pl.ds(j * CW, CW)]
                         for j in range(NINE)]
                    sj = sr_v[e, pl.ds(0, CW)]
                    sb = sr_v[e, pl.ds(CW, CW)]
                    vjs = [sr_v[e, pl.ds(2 * CW + kk * CW, CW)] for kk in range(3)]
                    vbs = [sr_v[e, pl.ds(5 * CW + kk * CW, CW)] for kk in range(3)]
                    vjd = [dr_v[e, pl.ds(kk * CW, CW)] for kk in range(3)]
                    vbd = [dr_v[e, pl.ds(3 * CW + kk * CW, CW)] for kk in range(3)]
                    msg_v[e, pl.ds(0, CW)] = s[0] * sj
                    msg_v[e, pl.ds(CW, CW)] = (vjs[0] * vbd[0]
                                               + vjs[1] * vbd[1]
                                               + vjs[2] * vbd[2])
                    for kk in range(3):
                        k1 = (kk + 1) % 3
                        k2 = (kk + 2) % 3
                        uk = w_v[e, pl.ds(NINE * CW + kk * CW, CW)]
                        cr_a = vjs[k1] * vbd[k2] - vjs[k2] * vbd[k1]
                        cr_b = vjs[k1] * vjd[k2] - vjs[k2] * vjd[k1]
                        cr_c = vbs[k1] * vbd[k2] - vbs[k2] * vbd[k1]
                        msg_v[e, pl.ds(2 * CW + kk * CW, CW)] = (
                            s[1] * uk + s[2] * vjd[kk] + s[3] * cr_a
                            + s[4] * sb * vbd[kk])
                        msg_v[e, pl.ds(5 * CW + kk * CW, CW)] = (
                            s[5] * vbd[kk] + s[6] * sb * vjd[kk]
                            + s[7] * cr_b + s[8] * cr_c)
                    return ecarry

                lax.fori_loop(0, EB, edge_body, 0)
                pltpu.sync_copy(msg_v, acc.at[cur[2]], add=True)
                finish_idx(icps, nxt, c)
              return bcarry

            lax.fori_loop(0, nblk // 2, block_body, 0)
            plsc.subcore_barrier()
            pltpu.sync_copy(acc.at[pl.ds(r0, ROWS_PER_TILE)],
                            out_h.at[cid, c, pl.ds(r0, ROWS_PER_TILE)])
            plsc.subcore_barrier()
            return carry

        lax.fori_loop(0, NCH, chunk_body, 0)

    return k(inv_c, w_c, srctab, dstaux, src8, dst8, sscat, zeros)


def kernel(s_j, sbar_j, v_j, vbar_j, r_ij, nbrs, W1, b1, W2, b2, Wd, bd):
    E = r_ij.shape[0]
    step = NW * EB * 2 * 512 // np.gcd(NW * EB * 2, 512)
    epad = ((E + step - 1) // step) * step

    W2p = W2.reshape(F, NINE, NCH, CW).transpose(0, 2, 1, 3)
    W2pp = jnp.pad(W2p.reshape(F, NCH, NINE * CW),
                   ((0, 0), (0, 0), (0, CROW - NINE * CW))).reshape(F, NCH * CROW)
    b2p = b2.reshape(NINE, NCH, CW).transpose(1, 0, 2)
    b2pp = jnp.pad(b2p.reshape(NCH, NINE * CW),
                   ((0, 0), (0, CROW - NINE * CW))).reshape(NCH * CROW)
    Wdp = Wd.reshape(NRBF, NINE, NCH, CW).transpose(2, 0, 1, 3).reshape(
        NCH, NRBF, NINE * CW)
    bdp = bd.reshape(NINE, NCH, CW).transpose(1, 0, 2).reshape(
        NCH, 1, NINE * CW)

    s_pad = jnp.pad(s_j, ((0, NODE_PAD - N), (0, 0)))
    inv = _node_mlp(s_pad, W1, b1, W2pp, b2pp)
    inv_c = inv.reshape(NODE_PAD * NCH, CROW)

    r_pad = jnp.pad(r_ij, ((0, epad - E), (0, 0)))
    w_c = _edge_weights(r_pad, Wdp, bdp)

    vj_c = v_j.reshape(N, NCH, CW, 3).transpose(0, 1, 3, 2).reshape(N * NCH, 3 * CW)
    vb_c = vbar_j.reshape(N, NCH, CW, 3).transpose(0, 1, 3, 2).reshape(N * NCH, 3 * CW)
    srctab = jnp.concatenate(
        [s_j.reshape(N * NCH, CW), sbar_j.reshape(N * NCH, CW), vj_c, vb_c],
        axis=1)
    dstaux = jnp.concatenate(
        [vj_c, vb_c, jnp.zeros((N * NCH, F - 6 * CW), jnp.float32)],
        axis=1)

    src = nbrs[:, 0].astype(jnp.int32)
    dst = nbrs[:, 1].astype(jnp.int32)
    src8 = jnp.pad(src * NCH, (0, epad + EB - E))
    dst8 = jnp.pad(dst * NCH, (0, epad + EB - E))
    sscat = jnp.pad(src, (0, epad + EB - E), constant_values=N)
    zeros = jnp.zeros((NPADN, F), jnp.float32)

    out = _sc_messages(inv_c, w_c, srctab, dstaux,
                       src8, dst8, sscat, zeros)

    o = (out[0] + out[1])[:, :N, :]
    dh = o[:, :, 0:CW].transpose(1, 0, 2).reshape(N, F)
    dhbar = o[:, :, CW:2 * CW].transpose(1, 0, 2).reshape(N, F)
    dv = (o[:, :, 2 * CW:5 * CW].reshape(NCH, N, 3, CW)
          .transpose(1, 0, 3, 2).reshape(N, F, 3))
    dvbar = (o[:, :, 5 * CW:8 * CW].reshape(NCH, N, 3, CW)
             .transpose(1, 0, 3, 2).reshape(N, F, 3))
    return (dh, dhbar, dv, dvbar)


# speedup vs baseline: 1.1440x; 1.1440x over previous
import functools
import numpy as np
import jax
import jax.numpy as jnp
from jax import lax
from jax.experimental import pallas as pl
from jax.experimental.pallas import tpu as pltpu
from jax.experimental.pallas import tpu_sc as plsc

N = 10000
F = 128
NRBF = 20
CUTOFF = 5.0
NCH = 8
CW = 16
NINE = 9
CROW = 256

NC, NS, NW = 2, 16, 32
EB = 48
NPADN = 10112
ROWS_PER_TILE = NPADN // NS
NODE_PAD = 10240


def _node_mlp(s_pad, W1, b1, W2pp, b2pp):
    nb = s_pad.shape[0] // 256

    def body(s_ref, w1_ref, b1_ref, w2_ref, b2_ref, o_ref):
        h = jax.nn.silu(jnp.dot(s_ref[...], w1_ref[...],
                                preferred_element_type=jnp.float32) + b1_ref[...])
        o_ref[...] = jnp.dot(h, w2_ref[...],
                             preferred_element_type=jnp.float32) + b2_ref[...]

    return pl.pallas_call(
        body,
        grid=(nb,),
        in_specs=[
            pl.BlockSpec((256, F), lambda i: (i, 0)),
            pl.BlockSpec((F, F), lambda i: (0, 0)),
            pl.BlockSpec((1, F), lambda i: (0, 0)),
            pl.BlockSpec((F, NCH * CROW), lambda i: (0, 0)),
            pl.BlockSpec((1, NCH * CROW), lambda i: (0, 0)),
        ],
        out_specs=pl.BlockSpec((256, NCH * CROW), lambda i: (i, 0)),
        out_shape=jax.ShapeDtypeStruct((s_pad.shape[0], NCH * CROW), jnp.float32),
    )(s_pad, W1, b1.reshape(1, F), W2pp, b2pp.reshape(1, NCH * CROW))


def _edge_weights(r_pad, Wdp, bdp):
    epad = r_pad.shape[0]
    nb = epad // 512

    def body(r_ref, wd_ref, bd_ref, w_ref):
        r = r_ref[...]
        d = jnp.sqrt(jnp.sum(r * r + 1e-8, axis=1, keepdims=True))
        u3 = r / d
        u48 = jnp.concatenate(
            [jnp.broadcast_to(u3[:, kk:kk + 1], (512, CW)) for kk in range(3)],
            axis=1)
        nline = (lax.broadcasted_iota(jnp.int32, (512, NRBF), 1) + 1
                 ).astype(jnp.float32)
        rbf = jnp.sin(nline * (np.pi / CUTOFF) * d) / d
        fcut = 0.5 * (jnp.cos(d * (np.pi / CUTOFF)) + 1.0)
        fcut = jnp.where(d < CUTOFF, fcut, 0.0)
        pad = jnp.zeros((512, CROW - NINE * CW - 3 * CW), jnp.float32)
        for c in range(NCH):
            wm = (jnp.dot(rbf, wd_ref[c],
                          preferred_element_type=jnp.float32)
                  + bd_ref[c]) * fcut
            w_ref[c] = jnp.concatenate([wm, u48, pad], axis=1)

    return pl.pallas_call(
        body,
        grid=(nb,),
        in_specs=[
            pl.BlockSpec((512, 3), lambda i: (i, 0)),
            pl.BlockSpec((NCH, NRBF, NINE * CW), lambda i: (0, 0, 0)),
            pl.BlockSpec((NCH, 1, NINE * CW), lambda i: (0, 0, 0)),
        ],
        out_specs=pl.BlockSpec((NCH, 512, CROW), lambda i: (0, i, 0)),
        out_shape=jax.ShapeDtypeStruct((NCH, epad, CROW), jnp.float32),
    )(r_pad, Wdp, bdp)


def _sc_messages(inv_c, w_c, srctab, dstaux, src8, dst8, sscat, zeros):
    epad = w_c.shape[1]
    per_tile = epad // NW
    nblk = per_tile // EB
    mesh = plsc.VectorSubcoreMesh(core_axis_name="c", subcore_axis_name="s")

    @functools.partial(
        pl.kernel,
        mesh=mesh,
        out_type=jax.ShapeDtypeStruct((NC, NCH, NPADN, F), jnp.float32),
        scratch_types=[
            pltpu.VMEM((EB, CROW), jnp.float32),
            pltpu.VMEM((EB, CROW), jnp.float32),
            pltpu.VMEM((EB, F), jnp.float32),
            pltpu.VMEM((EB, F), jnp.float32),
            pltpu.VMEM((EB, F), jnp.float32),
            pltpu.VMEM((EB,), jnp.int32),
            pltpu.VMEM((EB,), jnp.int32),
            pltpu.VMEM((EB,), jnp.int32),
            pltpu.VMEM((EB,), jnp.int32),
            pltpu.VMEM((EB,), jnp.int32),
            pltpu.SemaphoreType.DMA,
            pltpu.VMEM_SHARED((NPADN, F), jnp.float32),
        ],
    )
    def k(inv_h, w_h, st_h, dt_h, s8_h, d8_h, ss_h, z_h,
          out_h, inv_v, w_v, sr_v, dr_v, msg_v, s8_v, d8_v, si_v, di_v, ss_v,
          sem, acc):
        cid = lax.axis_index("c")
        sid = lax.axis_index("s")
        wid = cid * NS + sid
        e_base = wid * per_tile
        r0 = sid * ROWS_PER_TILE

        def chunk_body(c, carry):
            pltpu.sync_copy(z_h.at[pl.ds(r0, ROWS_PER_TILE)],
                            acc.at[pl.ds(r0, ROWS_PER_TILE)])
            plsc.subcore_barrier()

            def block_body(b, bcarry):
                e0 = e_base + b * EB
                cps = [pltpu.async_copy(s8_h.at[pl.ds(e0, EB)], s8_v, sem),
                       pltpu.async_copy(d8_h.at[pl.ds(e0, EB)], d8_v, sem),
                       pltpu.async_copy(ss_h.at[pl.ds(e0, EB)], ss_v, sem),
                       pltpu.async_copy(w_h.at[c, pl.ds(e0, EB)], w_v, sem)]
                for cp in cps[:3]:
                    cp.wait()
                for j in range(EB // 16):
                    sl = pl.ds(j * 16, 16)
                    si_v[sl] = s8_v[sl] + c
                    di_v[sl] = d8_v[sl] + c
                gps = [pltpu.async_copy(inv_h.at[di_v], inv_v, sem),
                       pltpu.async_copy(st_h.at[si_v], sr_v, sem),
                       pltpu.async_copy(dt_h.at[di_v], dr_v, sem)]
                cps[3].wait()
                for gp in gps:
                    gp.wait()

                def edge_body(e, ecarry):
                    s = [inv_v[e, pl.ds(j * CW, CW)] * w_v[e, pl.ds(j * CW, CW)]
                         for j in range(NINE)]
                    sj = sr_v[e, pl.ds(0, CW)]
                    sb = sr_v[e, pl.ds(CW, CW)]
                    vjs = [sr_v[e, pl.ds(2 * CW + kk * CW, CW)] for kk in range(3)]
                    vbs = [sr_v[e, pl.ds(5 * CW + kk * CW, CW)] for kk in range(3)]
                    vjd = [dr_v[e, pl.ds(kk * CW, CW)] for kk in range(3)]
                    vbd = [dr_v[e, pl.ds(3 * CW + kk * CW, CW)] for kk in range(3)]
                    msg_v[e, pl.ds(0, CW)] = s[0] * sj
                    msg_v[e, pl.ds(CW, CW)] = (vjs[0] * vbd[0]
                                               + vjs[1] * vbd[1]
                                               + vjs[2] * vbd[2])
                    for kk in range(3):
                        k1 = (kk + 1) % 3
                        k2 = (kk + 2) % 3
                        uk = w_v[e, pl.ds(NINE * CW + kk * CW, CW)]
                        cr_a = vjs[k1] * vbd[k2] - vjs[k2] * vbd[k1]
                        cr_b = vjs[k1] * vjd[k2] - vjs[k2] * vjd[k1]
                        cr_c = vbs[k1] * vbd[k2] - vbs[k2] * vbd[k1]
                        msg_v[e, pl.ds(2 * CW + kk * CW, CW)] = (
                            s[1] * uk + s[2] * vjd[kk] + s[3] * cr_a
                            + s[4] * sb * vbd[kk])
                        msg_v[e, pl.ds(5 * CW + kk * CW, CW)] = (
                            s[5] * vbd[kk] + s[6] * sb * vjd[kk]
                            + s[7] * cr_b + s[8] * cr_c)
                    return ecarry

                lax.fori_loop(0, EB, edge_body, 0)
                pltpu.sync_copy(msg_v, acc.at[ss_v], add=True)
                return bcarry

            lax.fori_loop(0, nblk, block_body, 0)
            plsc.subcore_barrier()
            pltpu.sync_copy(acc.at[pl.ds(r0, ROWS_PER_TILE)],
                            out_h.at[cid, c, pl.ds(r0, ROWS_PER_TILE)])
            plsc.subcore_barrier()
            return carry

        lax.fori_loop(0, NCH, chunk_body, 0)

    return k(inv_c, w_c, srctab, dstaux, src8, dst8, sscat, zeros)


def kernel(s_j, sbar_j, v_j, vbar_j, r_ij, nbrs, W1, b1, W2, b2, Wd, bd):
    E = r_ij.shape[0]
    step = NW * EB * 512 // np.gcd(NW * EB, 512)
    epad = ((E + step - 1) // step) * step

    W2p = W2.reshape(F, NINE, NCH, CW).transpose(0, 2, 1, 3)
    W2pp = jnp.pad(W2p.reshape(F, NCH, NINE * CW),
                   ((0, 0), (0, 0), (0, CROW - NINE * CW))).reshape(F, NCH * CROW)
    b2p = b2.reshape(NINE, NCH, CW).transpose(1, 0, 2)
    b2pp = jnp.pad(b2p.reshape(NCH, NINE * CW),
                   ((0, 0), (0, CROW - NINE * CW))).reshape(NCH * CROW)
    Wdp = Wd.reshape(NRBF, NINE, NCH, CW).transpose(2, 0, 1, 3).reshape(
        NCH, NRBF, NINE * CW)
    bdp = bd.reshape(NINE, NCH, CW).transpose(1, 0, 2).reshape(
        NCH, 1, NINE * CW)

    s_pad = jnp.pad(s_j, ((0, NODE_PAD - N), (0, 0)))
    inv = _node_mlp(s_pad, W1, b1, W2pp, b2pp)
    inv_c = inv.reshape(NODE_PAD * NCH, CROW)

    r_pad = jnp.pad(r_ij, ((0, epad - E), (0, 0)))
    w_c = _edge_weights(r_pad, Wdp, bdp)

    vj_c = v_j.reshape(N, NCH, CW, 3).transpose(0, 1, 3, 2).reshape(N * NCH, 3 * CW)
    vb_c = vbar_j.reshape(N, NCH, CW, 3).transpose(0, 1, 3, 2).reshape(N * NCH, 3 * CW)
    srctab = jnp.concatenate(
        [s_j.reshape(N * NCH, CW), sbar_j.reshape(N * NCH, CW), vj_c, vb_c],
        axis=1)
    dstaux = jnp.concatenate(
        [vj_c, vb_c, jnp.zeros((N * NCH, F - 6 * CW), jnp.float32)],
        axis=1)

    src = nbrs[:, 0].astype(jnp.int32)
    dst = nbrs[:, 1].astype(jnp.int32)
    src8 = jnp.pad(src * NCH, (0, epad + EB - E))
    dst8 = jnp.pad(dst * NCH, (0, epad + EB - E))
    sscat = jnp.pad(src, (0, epad + EB - E), constant_values=N)
    zeros = jnp.zeros((NPADN, F), jnp.float32)

    out = _sc_messages(inv_c, w_c, srctab, dstaux,
                       src8, dst8, sscat, zeros)

    o = (out[0] + out[1])[:, :N, :]
    dh = o[:, :, 0:CW].transpose(1, 0, 2).reshape(N, F)
    dhbar = o[:, :, CW:2 * CW].transpose(1, 0, 2).reshape(N, F)
    dv = (o[:, :, 2 * CW:5 * CW].reshape(NCH, N, 3, CW)
          .transpose(1, 0, 3, 2).reshape(N, F, 3))
    dvbar = (o[:, :, 5 * CW:8 * CW].reshape(NCH, N, 3, CW)
             .transpose(1, 0, 3, 2).reshape(N, F, 3))
    return (dh, dhbar, dv, dvbar)
